# baseline (speedup 1.0000x reference)
import jax
import jax.numpy as jnp
from jax.experimental import pallas as pl

VOCAB = 100000
SEED = 42
PRIOR_TOKENS = 4
DEPTH = 4
BATCH = 64
TOP_P = 0.9

BLOCK_V = 12800
BLOCK_B = 8


def _g_values(prior_ids):
    base = jax.random.key(SEED)

    def per_example(pid):
        kp = jax.random.fold_in(base, pid)
        rows = []
        for i in range(DEPTH):
            ki = jax.random.fold_in(kp, i)
            rows.append(jax.random.bernoulli(ki, 0.5, (VOCAB,)).astype(jnp.float32))
        return jnp.stack(rows, axis=0)

    return jax.vmap(per_example)(prior_ids)


def _scores_kernel(tok_ref, out_ref):
    j = pl.program_id(1)
    tok = tok_ref[...]
    col = jax.lax.broadcasted_iota(jnp.int32, (BLOCK_B, BLOCK_V), 1) + j * BLOCK_V
    out_ref[...] = jnp.where(col == tok, 100000.0, 1e-05)


def _assemble_scores(next_token):
    tok = next_token.reshape(BATCH, 1).astype(jnp.int32)
    grid = (BATCH // BLOCK_B, pl.cdiv(VOCAB, BLOCK_V))
    return pl.pallas_call(
        _scores_kernel,
        grid=grid,
        in_specs=[pl.BlockSpec((BLOCK_B, 1), lambda b, j: (b, 0))],
        out_specs=pl.BlockSpec((BLOCK_B, BLOCK_V), lambda b, j: (b, j)),
        out_shape=jax.ShapeDtypeStruct((BATCH, VOCAB), jnp.float32),
    )(tok)


def kernel(input_ids, logits):
    prior_ids = jnp.sum(input_ids[:, -PRIOR_TOKENS:], axis=-1).astype(jnp.int32)
    g_values = _g_values(prior_ids)
    probs = jax.nn.softmax(logits, axis=-1)
    for i in range(DEPTH):
        g_i = g_values[:, i, :]
        g_mass = jnp.sum(g_i * probs, axis=-1, keepdims=True)
        probs = probs * (1.0 + g_i - g_mass)
    order = jnp.argsort(-probs, axis=-1)
    sorted_probs = jnp.take_along_axis(probs, order, axis=-1)
    cum = jnp.cumsum(sorted_probs, axis=-1)
    cutoff = jax.vmap(lambda c: jnp.searchsorted(c, TOP_P, side='left'))(cum)
    keep = jnp.arange(VOCAB)[None, :] <= cutoff[:, None]
    sorted_probs = jnp.where(keep, sorted_probs, 0.0)
    denom = jnp.sum(sorted_probs, axis=-1, keepdims=True)
    sorted_probs = sorted_probs / denom
    sorted_probs = jnp.where(jnp.isfinite(sorted_probs), sorted_probs, 0.0)
    skeys = jax.vmap(lambda b: jax.random.fold_in(jax.random.key(SEED + 1), b))(
        jnp.arange(BATCH))
    sampled = jax.vmap(lambda k, p: jax.random.categorical(k, jnp.log(p)))(
        skeys, sorted_probs)
    next_token = jnp.take_along_axis(order, sampled[:, None], axis=-1)[:, 0]
    return _assemble_scores(next_token)


# speedup vs baseline: 1.0504x; 1.0504x over previous
import jax
import jax.numpy as jnp
from jax import lax
from jax.experimental import pallas as pl
from jax.experimental.pallas import tpu as pltpu

VOCAB = 100000
R = 100
C = 1000
SEED = 42
PRIOR_TOKENS = 4
DEPTH = 4
BATCH = 64
TOP_P = 0.9
TINY = 1.1754943508222875e-38


def _rotl(x, r):
    return lax.shift_left(x, jnp.int32(r)) | lax.shift_right_logical(
        x, jnp.int32(32 - r))


_ROTS = ((13, 15, 26, 6), (17, 29, 16, 24))


def _threefry_pair(k0, k1, x0, x1):
    ks2 = k0 ^ k1 ^ jnp.int32(0x1BD11BDA)
    ks = (k0, k1, ks2)
    x0 = x0 + k0
    x1 = x1 + k1
    for i in range(5):
        for r in _ROTS[i % 2]:
            x0 = x0 + x1
            x1 = _rotl(x1, r)
            x1 = x1 ^ x0
        x0 = x0 + ks[(i + 1) % 3]
        x1 = x1 + ks[(i + 2) % 3] + jnp.int32(i + 1)
    return x0, x1


def _gvals_kernel(pid_ref, o_ref):
    pid = pid_ref[0, 0, 0]
    kp0, kp1 = _threefry_pair(jnp.int32(0), jnp.int32(SEED), jnp.int32(0), pid)
    pos = lax.broadcasted_iota(jnp.int32, (R, C), 0) * C \
        + lax.broadcasted_iota(jnp.int32, (R, C), 1)
    nib = jnp.zeros((R, C), jnp.int32)
    for i in range(DEPTH):
        ki0, ki1 = _threefry_pair(kp0, kp1, jnp.int32(0), jnp.int32(i))
        o0, o1 = _threefry_pair(ki0, ki1, jnp.zeros_like(pos), pos)
        bits = o0 ^ o1
        nib = nib | jnp.where(bits >= 0, jnp.int32(1 << i), jnp.int32(0))
    o_ref[0] = nib


def _sample_kernel(s_ref, ord_ref, o_ref):
    b = pl.program_id(0)
    s = s_ref[0]
    ti = lax.broadcasted_iota(jnp.int32, (C, C), 0)
    tj = lax.broadcasted_iota(jnp.int32, (C, C), 1)
    T = (ti < tj).astype(jnp.float32)
    W = jnp.dot(s, T, preferred_element_type=jnp.float32)
    rs = jnp.sum(s, axis=1, keepdims=True)
    ui = lax.broadcasted_iota(jnp.int32, (R, R), 0)
    uj = lax.broadcasted_iota(jnp.int32, (R, R), 1)
    U = (uj < ui).astype(jnp.float32)
    Mrow = jnp.dot(U, rs, preferred_element_type=jnp.float32)
    E = W + Mrow
    keep = E < jnp.float32(TOP_P)
    sk = jnp.where(keep, s, 0.0)
    denom = jnp.sum(sk)
    lg = jnp.log(sk / denom)
    k0, k1 = _threefry_pair(jnp.int32(0), jnp.int32(SEED + 1), jnp.int32(0), b)
    pos = (lax.broadcasted_iota(jnp.int32, (R, C), 0) * C
           + lax.broadcasted_iota(jnp.int32, (R, C), 1))
    o0, o1 = _threefry_pair(k0, k1, jnp.zeros_like(pos), pos)
    bits = o0 ^ o1
    u9 = lax.shift_right_logical(bits, jnp.int32(9)) | jnp.int32(0x3F800000)
    u = lax.bitcast_convert_type(u9, jnp.float32) - 1.0
    uu = jnp.maximum(jnp.float32(TINY), u + jnp.float32(TINY))
    G = -jnp.log(-jnp.log(uu))
    val = lg + G
    mx = jnp.max(val)
    cand = jnp.where(val == mx, pos, jnp.int32(2**31 - 1))
    jstar = jnp.min(cand)
    tok = jnp.sum(jnp.where(pos == jstar, ord_ref[0], 0))
    o_ref[0] = jnp.where(pos == tok, 100000.0, 1e-05)


def kernel(input_ids, logits):
    pid = jnp.sum(input_ids[:, -PRIOR_TOKENS:], axis=-1).astype(jnp.int32)
    pid = pid.reshape(BATCH, 1, 1)

    nib = pl.pallas_call(
        _gvals_kernel,
        grid=(BATCH,),
        in_specs=[
            pl.BlockSpec((1, 1, 1), lambda b: (b, 0, 0),
                         memory_space=pltpu.SMEM),
        ],
        out_specs=pl.BlockSpec((1, R, C), lambda b: (b, 0, 0)),
        out_shape=jax.ShapeDtypeStruct((BATCH, R, C), jnp.int32),
    )(pid)
    nib = nib.reshape(BATCH, VOCAB)

    probs = jax.nn.softmax(logits, axis=-1)
    for i in range(DEPTH):
        g_i = ((nib >> i) & 1).astype(jnp.float32)
        g_mass = jnp.sum(g_i * probs, axis=-1, keepdims=True)
        probs = probs * (1.0 + g_i - g_mass)

    iota = lax.broadcasted_iota(jnp.int32, (BATCH, VOCAB), 1)
    sneg, order = lax.sort((-probs, iota), dimension=1, is_stable=True,
                           num_keys=1)
    sp = -sneg

    scores = pl.pallas_call(
        _sample_kernel,
        grid=(BATCH,),
        in_specs=[
            pl.BlockSpec((1, R, C), lambda b: (b, 0, 0)),
            pl.BlockSpec((1, R, C), lambda b: (b, 0, 0)),
        ],
        out_specs=pl.BlockSpec((1, R, C), lambda b: (b, 0, 0)),
        out_shape=jax.ShapeDtypeStruct((BATCH, R, C), jnp.float32),
    )(sp.reshape(BATCH, R, C), order.reshape(BATCH, R, C))
    return scores.reshape(BATCH, VOCAB)


# speedup vs baseline: 1.7385x; 1.6550x over previous
import jax
import jax.numpy as jnp
from jax import lax
from jax.experimental import pallas as pl
from jax.experimental.pallas import tpu as pltpu

VOCAB = 100000
R = 100
C = 1000
SEED = 42
PRIOR_TOKENS = 4
DEPTH = 4
BATCH = 64
TOP_P = 0.9
TINY = 1.1754943508222875e-38


def _rotl(x, r):
    return lax.shift_left(x, jnp.int32(r)) | lax.shift_right_logical(
        x, jnp.int32(32 - r))


_ROTS = ((13, 15, 26, 6), (17, 29, 16, 24))


def _threefry_pair(k0, k1, x0, x1):
    ks2 = k0 ^ k1 ^ jnp.int32(0x1BD11BDA)
    ks = (k0, k1, ks2)
    x0 = x0 + k0
    x1 = x1 + k1
    for i in range(5):
        for r in _ROTS[i % 2]:
            x0 = x0 + x1
            x1 = _rotl(x1, r)
            x1 = x1 ^ x0
        x0 = x0 + ks[(i + 1) % 3]
        x1 = x1 + ks[(i + 2) % 3] + jnp.int32(i + 1)
    return x0, x1


def _pos_iota():
    return (lax.broadcasted_iota(jnp.int32, (R, C), 0) * C
            + lax.broadcasted_iota(jnp.int32, (R, C), 1))


def _gvals_kernel(pid_ref, o_ref):
    pid = pid_ref[0, 0, 0]
    kp0, kp1 = _threefry_pair(jnp.int32(0), jnp.int32(SEED), jnp.int32(0), pid)
    pos = _pos_iota()
    nib = jnp.zeros((R, C), jnp.int32)
    for i in range(DEPTH):
        ki0, ki1 = _threefry_pair(kp0, kp1, jnp.int32(0), jnp.int32(i))
        o0, o1 = _threefry_pair(ki0, ki1, jnp.zeros_like(pos), pos)
        bits = o0 ^ o1
        nib = nib | jnp.where(bits >= 0, jnp.int32(1 << i), jnp.int32(0))
    o_ref[0] = nib


def _sample_kernel(s_ref, sval_ref, cnt_ref):
    b = pl.program_id(0)
    s = s_ref[0]
    ti = lax.broadcasted_iota(jnp.int32, (C, C), 0)
    tj = lax.broadcasted_iota(jnp.int32, (C, C), 1)
    T = (ti < tj).astype(jnp.float32)
    W = jnp.dot(s, T, preferred_element_type=jnp.float32)
    rs = jnp.sum(s, axis=1, keepdims=True)
    ui = lax.broadcasted_iota(jnp.int32, (R, R), 0)
    uj = lax.broadcasted_iota(jnp.int32, (R, R), 1)
    U = (uj < ui).astype(jnp.float32)
    Mrow = jnp.dot(U, rs, preferred_element_type=jnp.float32)
    E = W + Mrow
    keep = E < jnp.float32(TOP_P)
    sk = jnp.where(keep, s, 0.0)
    denom = jnp.sum(sk)
    lg = jnp.log(sk / denom)
    k0, k1 = _threefry_pair(jnp.int32(0), jnp.int32(SEED + 1), jnp.int32(0), b)
    pos = _pos_iota()
    o0, o1 = _threefry_pair(k0, k1, jnp.zeros_like(pos), pos)
    bits = o0 ^ o1
    u9 = lax.shift_right_logical(bits, jnp.int32(9)) | jnp.int32(0x3F800000)
    u = lax.bitcast_convert_type(u9, jnp.float32) - 1.0
    uu = jnp.maximum(jnp.float32(TINY), u + jnp.float32(TINY))
    G = -jnp.log(-jnp.log(uu))
    val = lg + G
    mx = jnp.max(val)
    cand = jnp.where(val == mx, pos, jnp.int32(2**31 - 1))
    jstar = jnp.min(cand)
    sval = jnp.sum(jnp.where(pos == jstar, s, 0.0))
    cnt = jnp.sum(jnp.where((s == sval) & (pos < jstar), 1.0, 0.0))
    sval_ref[0, 0, 0] = sval
    cnt_ref[0, 0, 0] = cnt


def _scores_kernel(p_ref, sval_ref, cnt_ref, o_ref):
    p = p_ref[0]
    sval = sval_ref[0, 0, 0]
    cnt = cnt_ref[0, 0, 0]
    eq = (p == sval).astype(jnp.float32)
    ti = lax.broadcasted_iota(jnp.int32, (C, C), 0)
    tj = lax.broadcasted_iota(jnp.int32, (C, C), 1)
    T = (ti < tj).astype(jnp.float32)
    W = jnp.dot(eq, T, preferred_element_type=jnp.float32)
    rs = jnp.sum(eq, axis=1, keepdims=True)
    ui = lax.broadcasted_iota(jnp.int32, (R, R), 0)
    uj = lax.broadcasted_iota(jnp.int32, (R, R), 1)
    U = (uj < ui).astype(jnp.float32)
    Mrow = jnp.dot(U, rs, preferred_element_type=jnp.float32)
    occ = W + Mrow
    hit = (p == sval) & (occ == cnt)
    o_ref[0] = jnp.where(hit, 100000.0, 1e-05)


def kernel(input_ids, logits):
    pid = jnp.sum(input_ids[:, -PRIOR_TOKENS:], axis=-1).astype(jnp.int32)
    pid = pid.reshape(BATCH, 1, 1)

    nib = pl.pallas_call(
        _gvals_kernel,
        grid=(BATCH,),
        in_specs=[
            pl.BlockSpec((1, 1, 1), lambda b: (b, 0, 0),
                         memory_space=pltpu.SMEM),
        ],
        out_specs=pl.BlockSpec((1, R, C), lambda b: (b, 0, 0)),
        out_shape=jax.ShapeDtypeStruct((BATCH, R, C), jnp.int32),
    )(pid)
    nib = nib.reshape(BATCH, VOCAB)

    probs = jax.nn.softmax(logits, axis=-1)
    for i in range(DEPTH):
        g_i = ((nib >> i) & 1).astype(jnp.float32)
        g_mass = jnp.sum(g_i * probs, axis=-1, keepdims=True)
        probs = probs * (1.0 + g_i - g_mass)

    sp = -lax.sort(-probs, dimension=1, is_stable=False)

    sval, cnt = pl.pallas_call(
        _sample_kernel,
        grid=(BATCH,),
        in_specs=[pl.BlockSpec((1, R, C), lambda b: (b, 0, 0))],
        out_specs=[
            pl.BlockSpec((1, 1, 1), lambda b: (b, 0, 0),
                         memory_space=pltpu.SMEM),
            pl.BlockSpec((1, 1, 1), lambda b: (b, 0, 0),
                         memory_space=pltpu.SMEM),
        ],
        out_shape=[
            jax.ShapeDtypeStruct((BATCH, 1, 1), jnp.float32),
            jax.ShapeDtypeStruct((BATCH, 1, 1), jnp.float32),
        ],
    )(sp.reshape(BATCH, R, C))

    scores = pl.pallas_call(
        _scores_kernel,
        grid=(BATCH,),
        in_specs=[
            pl.BlockSpec((1, R, C), lambda b: (b, 0, 0)),
            pl.BlockSpec((1, 1, 1), lambda b: (b, 0, 0),
                         memory_space=pltpu.SMEM),
            pl.BlockSpec((1, 1, 1), lambda b: (b, 0, 0),
                         memory_space=pltpu.SMEM),
        ],
        out_specs=pl.BlockSpec((1, R, C), lambda b: (b, 0, 0)),
        out_shape=jax.ShapeDtypeStruct((BATCH, R, C), jnp.float32),
    )(probs.reshape(BATCH, R, C), sval, cnt)
    return scores.reshape(BATCH, VOCAB)


# speedup vs baseline: 1.7573x; 1.0108x over previous
import jax
import jax.numpy as jnp
from jax import lax
from jax.experimental import pallas as pl
from jax.experimental.pallas import tpu as pltpu

VOCAB = 100000
R = 100
C = 1000
SEED = 42
PRIOR_TOKENS = 4
DEPTH = 4
BATCH = 64
TOP_P = 0.9
TINY = 1.1754943508222875e-38


def _rotl(x, r):
    return lax.shift_left(x, jnp.int32(r)) | lax.shift_right_logical(
        x, jnp.int32(32 - r))


_ROTS = ((13, 15, 26, 6), (17, 29, 16, 24))


def _threefry_pair(k0, k1, x0, x1):
    ks2 = k0 ^ k1 ^ jnp.int32(0x1BD11BDA)
    ks = (k0, k1, ks2)
    x0 = x0 + k0
    x1 = x1 + k1
    for i in range(5):
        for r in _ROTS[i % 2]:
            x0 = x0 + x1
            x1 = _rotl(x1, r)
            x1 = x1 ^ x0
        x0 = x0 + ks[(i + 1) % 3]
        x1 = x1 + ks[(i + 2) % 3] + jnp.int32(i + 1)
    return x0, x1


def _pos_iota():
    return (lax.broadcasted_iota(jnp.int32, (R, C), 0) * C
            + lax.broadcasted_iota(jnp.int32, (R, C), 1))


def _gvals_kernel(pid_ref, o_ref):
    pid = pid_ref[0, 0, 0]
    kp0, kp1 = _threefry_pair(jnp.int32(0), jnp.int32(SEED), jnp.int32(0), pid)
    pos = _pos_iota()
    nib = jnp.zeros((R, C), jnp.int32)
    for i in range(DEPTH):
        ki0, ki1 = _threefry_pair(kp0, kp1, jnp.int32(0), jnp.int32(i))
        o0, o1 = _threefry_pair(ki0, ki1, jnp.zeros_like(pos), pos)
        bits = o0 ^ o1
        nib = nib | jnp.where(bits >= 0, jnp.int32(1 << i), jnp.int32(0))
    o_ref[0] = nib


def _final_kernel(sneg_ref, p_ref, o_ref):
    b = pl.program_id(0)
    s = -sneg_ref[0]
    ti = lax.broadcasted_iota(jnp.int32, (C, C), 0)
    tj = lax.broadcasted_iota(jnp.int32, (C, C), 1)
    T = (ti < tj).astype(jnp.float32)
    W = jnp.dot(s, T, preferred_element_type=jnp.float32)
    rs = jnp.sum(s, axis=1, keepdims=True)
    ui = lax.broadcasted_iota(jnp.int32, (R, R), 0)
    uj = lax.broadcasted_iota(jnp.int32, (R, R), 1)
    U = (uj < ui).astype(jnp.float32)
    Mrow = jnp.dot(U, rs, preferred_element_type=jnp.float32)
    E = W + Mrow
    keep = E < jnp.float32(TOP_P)
    sk = jnp.where(keep, s, 0.0)
    denom = jnp.sum(sk)
    lg = jnp.log(sk / denom)
    k0, k1 = _threefry_pair(jnp.int32(0), jnp.int32(SEED + 1), jnp.int32(0), b)
    pos = _pos_iota()
    o0, o1 = _threefry_pair(k0, k1, jnp.zeros_like(pos), pos)
    bits = o0 ^ o1
    u9 = lax.shift_right_logical(bits, jnp.int32(9)) | jnp.int32(0x3F800000)
    u = lax.bitcast_convert_type(u9, jnp.float32) - 1.0
    uu = jnp.maximum(jnp.float32(TINY), u + jnp.float32(TINY))
    G = -jnp.log(-jnp.log(uu))
    val = lg + G
    mx = jnp.max(val)
    cand = jnp.where(val == mx, pos, jnp.int32(2**31 - 1))
    jstar = jnp.min(cand)
    sval = jnp.sum(jnp.where(pos == jstar, s, 0.0))
    cnt = jnp.sum(jnp.where((s == sval) & (pos < jstar), 1.0, 0.0))
    p = p_ref[0]
    eq = (p == sval).astype(jnp.float32)
    W2 = jnp.dot(eq, T, preferred_element_type=jnp.float32)
    rs2 = jnp.sum(eq, axis=1, keepdims=True)
    Mrow2 = jnp.dot(U, rs2, preferred_element_type=jnp.float32)
    occ = W2 + Mrow2
    hit = (p == sval) & (occ == cnt)
    o_ref[0] = jnp.where(hit, 100000.0, 1e-05)


def kernel(input_ids, logits):
    pid = jnp.sum(input_ids[:, -PRIOR_TOKENS:], axis=-1).astype(jnp.int32)
    pid = pid.reshape(BATCH, 1, 1)

    nib = pl.pallas_call(
        _gvals_kernel,
        grid=(BATCH,),
        in_specs=[
            pl.BlockSpec((1, 1, 1), lambda b: (b, 0, 0),
                         memory_space=pltpu.SMEM),
        ],
        out_specs=pl.BlockSpec((1, R, C), lambda b: (b, 0, 0)),
        out_shape=jax.ShapeDtypeStruct((BATCH, R, C), jnp.int32),
    )(pid)
    nib = nib.reshape(BATCH, VOCAB)

    probs = jax.nn.softmax(logits, axis=-1)
    for i in range(DEPTH):
        g_i = ((nib >> i) & 1).astype(jnp.float32)
        g_mass = jnp.sum(g_i * probs, axis=-1, keepdims=True)
        probs = probs * (1.0 + g_i - g_mass)

    sneg = lax.sort(-probs, dimension=1, is_stable=False)

    scores = pl.pallas_call(
        _final_kernel,
        grid=(BATCH,),
        in_specs=[
            pl.BlockSpec((1, R, C), lambda b: (b, 0, 0)),
            pl.BlockSpec((1, R, C), lambda b: (b, 0, 0)),
        ],
        out_specs=pl.BlockSpec((1, R, C), lambda b: (b, 0, 0)),
        out_shape=jax.ShapeDtypeStruct((BATCH, R, C), jnp.float32),
    )(sneg.reshape(BATCH, R, C), probs.reshape(BATCH, R, C))
    return scores.reshape(BATCH, VOCAB)


# speedup vs baseline: 2.6080x; 1.4841x over previous
import jax
import jax.numpy as jnp
from jax import lax
from jax.experimental import pallas as pl
from jax.experimental.pallas import tpu as pltpu

VOCAB = 100000
R = 100
C = 1000
SEED = 42
PRIOR_TOKENS = 4
DEPTH = 4
BATCH = 64
TOP_P = 0.9
TINY = 1.1754943508222875e-38


def _rotl(x, r):
    return lax.shift_left(x, jnp.int32(r)) | lax.shift_right_logical(
        x, jnp.int32(32 - r))


_ROTS = ((13, 15, 26, 6), (17, 29, 16, 24))


def _threefry_pair(k0, k1, x0, x1):
    ks2 = k0 ^ k1 ^ jnp.int32(0x1BD11BDA)
    ks = (k0, k1, ks2)
    x0 = x0 + k0
    x1 = x1 + k1
    for i in range(5):
        for r in _ROTS[i % 2]:
            x0 = x0 + x1
            x1 = _rotl(x1, r)
            x1 = x1 ^ x0
        x0 = x0 + ks[(i + 1) % 3]
        x1 = x1 + ks[(i + 2) % 3] + jnp.int32(i + 1)
    return x0, x1


def _pos_iota():
    return (lax.broadcasted_iota(jnp.int32, (R, C), 0) * C
            + lax.broadcasted_iota(jnp.int32, (R, C), 1))


def _gvals_kernel(pid_ref, o_ref):
    pid = pid_ref[0, 0, 0]
    kp0, kp1 = _threefry_pair(jnp.int32(0), jnp.int32(SEED), jnp.int32(0), pid)
    pos = _pos_iota()
    nib = jnp.zeros((R, C), jnp.int32)
    for i in range(DEPTH):
        ki0, ki1 = _threefry_pair(kp0, kp1, jnp.int32(0), jnp.int32(i))
        o0, o1 = _threefry_pair(ki0, ki1, jnp.zeros_like(pos), pos)
        bits = o0 ^ o1
        nib = nib | jnp.where(bits >= 0, jnp.int32(1 << i), jnp.int32(0))
    o_ref[0] = nib


def _final_kernel(sneg_ref, p_ref, o_ref):
    b = pl.program_id(0)
    s = lax.bitcast_convert_type(-sneg_ref[0], jnp.float32)
    ti = lax.broadcasted_iota(jnp.int32, (C, C), 0)
    tj = lax.broadcasted_iota(jnp.int32, (C, C), 1)
    T = (ti < tj).astype(jnp.float32)
    W = jnp.dot(s, T, preferred_element_type=jnp.float32)
    rs = jnp.sum(s, axis=1, keepdims=True)
    ui = lax.broadcasted_iota(jnp.int32, (R, R), 0)
    uj = lax.broadcasted_iota(jnp.int32, (R, R), 1)
    U = (uj < ui).astype(jnp.float32)
    Mrow = jnp.dot(U, rs, preferred_element_type=jnp.float32)
    E = W + Mrow
    keep = E < jnp.float32(TOP_P)
    sk = jnp.where(keep, s, 0.0)
    denom = jnp.sum(sk)
    lg = jnp.log(sk / denom)
    k0, k1 = _threefry_pair(jnp.int32(0), jnp.int32(SEED + 1), jnp.int32(0), b)
    pos = _pos_iota()
    o0, o1 = _threefry_pair(k0, k1, jnp.zeros_like(pos), pos)
    bits = o0 ^ o1
    u9 = lax.shift_right_logical(bits, jnp.int32(9)) | jnp.int32(0x3F800000)
    u = lax.bitcast_convert_type(u9, jnp.float32) - 1.0
    uu = jnp.maximum(jnp.float32(TINY), u + jnp.float32(TINY))
    G = -jnp.log(-jnp.log(uu))
    val = lg + G
    mx = jnp.max(val)
    cand = jnp.where(val == mx, pos, jnp.int32(2**31 - 1))
    jstar = jnp.min(cand)
    sval = jnp.sum(jnp.where(pos == jstar, s, 0.0))
    cnt = jnp.sum(jnp.where((s == sval) & (pos < jstar), 1.0, 0.0))
    p = p_ref[0]
    eq = (p == sval).astype(jnp.float32)
    W2 = jnp.dot(eq, T, preferred_element_type=jnp.float32)
    rs2 = jnp.sum(eq, axis=1, keepdims=True)
    Mrow2 = jnp.dot(U, rs2, preferred_element_type=jnp.float32)
    occ = W2 + Mrow2
    hit = (p == sval) & (occ == cnt)
    o_ref[0] = jnp.where(hit, 100000.0, 1e-05)


def kernel(input_ids, logits):
    pid = jnp.sum(input_ids[:, -PRIOR_TOKENS:], axis=-1).astype(jnp.int32)
    pid = pid.reshape(BATCH, 1, 1)

    nib = pl.pallas_call(
        _gvals_kernel,
        grid=(BATCH,),
        in_specs=[
            pl.BlockSpec((1, 1, 1), lambda b: (b, 0, 0),
                         memory_space=pltpu.SMEM),
        ],
        out_specs=pl.BlockSpec((1, R, C), lambda b: (b, 0, 0)),
        out_shape=jax.ShapeDtypeStruct((BATCH, R, C), jnp.int32),
    )(pid)
    nib = nib.reshape(BATCH, VOCAB)

    probs = jax.nn.softmax(logits, axis=-1)
    for i in range(DEPTH):
        g_i = ((nib >> i) & 1).astype(jnp.float32)
        g_mass = jnp.sum(g_i * probs, axis=-1, keepdims=True)
        probs = probs * (1.0 + g_i - g_mass)

    pb = lax.bitcast_convert_type(probs, jnp.int32)
    sneg = lax.sort(-pb, dimension=1, is_stable=False)

    scores = pl.pallas_call(
        _final_kernel,
        grid=(BATCH,),
        in_specs=[
            pl.BlockSpec((1, R, C), lambda b: (b, 0, 0)),
            pl.BlockSpec((1, R, C), lambda b: (b, 0, 0)),
        ],
        out_specs=pl.BlockSpec((1, R, C), lambda b: (b, 0, 0)),
        out_shape=jax.ShapeDtypeStruct((BATCH, R, C), jnp.float32),
    )(sneg.reshape(BATCH, R, C), probs.reshape(BATCH, R, C))
    return scores.reshape(BATCH, VOCAB)
